# baseline (speedup 1.0000x reference)
import jax
import jax.numpy as jnp
from jax import lax
from jax.experimental import pallas as pl
from jax.experimental.pallas import tpu as pltpu
from jax.experimental.pallas import tpu_sc as plsc

DIM = 64
NUM_CTX = 5
NC = 2
NS = 16
NW = NC * NS
CB = 128
VB = 2048
HB = VB // 2


QB = VB // 4
WPR = DIM // 2


def _tc_pack_body(x_ref, o_ref):
    r = lax.broadcasted_iota(jnp.int32, (WPR, WPR), 0)
    c = lax.broadcasted_iota(jnp.int32, (WPR, WPR), 1)
    eye = (r == c).astype(jnp.bfloat16)

    def tpose(xb):
        return lax.dot_general(xb, eye, (((0,), (0,)), ((), ())),
                               preferred_element_type=jnp.float32)

    lo = tpose(x_ref[0:WPR, :].astype(jnp.bfloat16))
    hi = tpose(x_ref[WPR:DIM, :].astype(jnp.bfloat16))
    lu = lax.bitcast_convert_type(lo, jnp.uint32) >> 16
    hu = lax.bitcast_convert_type(hi, jnp.uint32)
    wv = lax.bitcast_convert_type(lu | hu, jnp.float32)
    for q in range(4):
        o_ref[:, WPR * q:WPR * (q + 1)] = wv[QB * q:QB * (q + 1), :]


def _tc_pack(ct, grid):
    return pl.pallas_call(
        _tc_pack_body,
        grid=(grid,),
        in_specs=[pl.BlockSpec((DIM, VB), lambda i: (0, i))],
        out_specs=pl.BlockSpec((QB, 4 * WPR), lambda i: (i, 0)),
        out_shape=jax.ShapeDtypeStruct((grid * QB, 4 * WPR), jnp.float32),
        compiler_params=pltpu.CompilerParams(
            dimension_semantics=("arbitrary",)),
    )(ct)


def _make_sc_body(nchunk):
    def body(tgt_i, ctx_i, tgt_tab, ctx_tab, out,
             tgt_idx_v, ctx_idx_v,
             tgt_a, tgt_b, ctx_a, ctx_b, out_v, sem_a, sem_b):
        w = lax.axis_index("s") * NC + lax.axis_index("c")
        lanes = lax.iota(jnp.int32, 16)
        pltpu.sync_copy(tgt_i.at[w], tgt_idx_v)
        pltpu.sync_copy(ctx_i.at[w], ctx_idx_v)

        tgt_bufs = (tgt_a, tgt_b)
        ctx_bufs = (ctx_a, ctx_b)
        sems = (sem_a, sem_b)

        def fire(k):
            par = k % 2
            waits = [pltpu.async_copy(
                tgt_tab.at[tgt_idx_v.at[k]], tgt_bufs[par], sems[par])]
            for c in range(NUM_CTX):
                waits.append(pltpu.async_copy(
                    ctx_tab.at[ctx_idx_v.at[k, c]],
                    ctx_bufs[par].at[pl.ds(c * CB, CB)], sems[par]))
            return waits

        def unpack2(row_ref, r):
            a0, b0 = plsc.unpack(plsc.bitcast(row_ref[r, pl.ds(0, 16)],
                                              jnp.bfloat16),
                                 format=plsc.PackFormat.INTERLEAVED)
            a1, b1 = plsc.unpack(plsc.bitcast(row_ref[r, pl.ds(16, 16)],
                                              jnp.bfloat16),
                                 format=plsc.PackFormat.INTERLEAVED)
            return (a0, a1, b0, b1)

        pending = fire(0)
        for k in range(nchunk):
            for h in pending:
                h.wait()
            if k + 1 < nchunk:
                pending = fire(k + 1)
            tgt_rows = tgt_bufs[k % 2]
            ctx_rows = ctx_bufs[k % 2]
            ks = jnp.full((16,), k, jnp.int32)

            def bstep(b, carry, k=k, tgt_rows=tgt_rows, ctx_rows=ctx_rows,
                      ks=ks):
                wv = [tgt_rows[b, pl.ds(16 * i, 16)] for i in range(4)]
                vec = jnp.zeros((16,), jnp.float32)
                for s in range(NUM_CTX):
                    p = b * NUM_CTX + s
                    xv = unpack2(ctx_rows, p)
                    acc = wv[0] * xv[0]
                    for i in range(1, 4):
                        acc = acc + wv[i] * xv[i]
                    vec = jnp.where(lanes == s, jnp.sum(acc), vec)
                plsc.store_scatter(out_v, [ks, b * NUM_CTX + lanes], vec,
                                   mask=lanes < NUM_CTX)
                return carry

            lax.fori_loop(0, CB, bstep, 0)

        pltpu.sync_copy(out_v, out.at[w])

    return body


def kernel(target, context, target_table, context_table):
    batch, num_ctx = context.shape
    vocab = target_table.shape[0]
    assert num_ctx == NUM_CTX and batch % (NW * CB) == 0
    nchunk = batch // (NW * CB)
    ppc = CB * NUM_CTX
    grid = (vocab + VB - 1) // VB

    cpk = _tc_pack(context_table.T, grid).reshape(4 * grid * QB, WPR)

    def addr(v):
        v = v.astype(jnp.int32)
        return (v >> 11) * (4 * QB) + (v & (QB - 1)) * 4 + ((v >> 9) & 3)

    tgt_i = target.astype(jnp.int32).reshape(NW, nchunk, CB)
    ctx_i = addr(context).reshape(NW, nchunk, NUM_CTX, CB)

    mesh = plsc.VectorSubcoreMesh(core_axis_name="c", subcore_axis_name="s")
    grid_kernel = pl.kernel(
        _make_sc_body(nchunk),
        out_type=jax.ShapeDtypeStruct((NW, nchunk, ppc), jnp.float32),
        mesh=mesh,
        scratch_types=[
            pltpu.VMEM((nchunk, CB), jnp.int32),
            pltpu.VMEM((nchunk, NUM_CTX, CB), jnp.int32),
            pltpu.VMEM((CB, DIM), jnp.float32),
            pltpu.VMEM((CB, DIM), jnp.float32),
            pltpu.VMEM((NUM_CTX * CB, WPR), jnp.float32),
            pltpu.VMEM((NUM_CTX * CB, WPR), jnp.float32),
            pltpu.VMEM((nchunk, ppc), jnp.float32),
            pltpu.SemaphoreType.DMA,
            pltpu.SemaphoreType.DMA,
        ],
        compiler_params=pltpu.CompilerParams(
            needs_layout_passes=False, use_tc_tiling_on_sc=False),
    )
    out = grid_kernel(tgt_i, ctx_i, target_table, cpk)
    return out.reshape(batch, NUM_CTX)


# speedup vs baseline: 1.2819x; 1.2819x over previous
import jax
import jax.numpy as jnp
from jax import lax
from jax.experimental import pallas as pl
from jax.experimental.pallas import tpu as pltpu
from jax.experimental.pallas import tpu_sc as plsc

DIM = 64
NUM_CTX = 5
NC = 2
NS = 16
NW = NC * NS
CB = 128
VB = 16384
HB = VB // 2


QB = VB // 4
WPR = DIM // 2


def _tc_pack_body(x_ref, o_ref):
    r = lax.broadcasted_iota(jnp.int32, (WPR, WPR), 0)
    c = lax.broadcasted_iota(jnp.int32, (WPR, WPR), 1)
    eye = (r == c).astype(jnp.bfloat16)

    def tpose(xb):
        return lax.dot_general(xb, eye, (((0,), (0,)), ((), ())),
                               preferred_element_type=jnp.float32)

    lo = tpose(x_ref[0:WPR, :].astype(jnp.bfloat16))
    hi = tpose(x_ref[WPR:DIM, :].astype(jnp.bfloat16))
    lu = lax.bitcast_convert_type(lo, jnp.uint32) >> 16
    hu = lax.bitcast_convert_type(hi, jnp.uint32)
    wv = lax.bitcast_convert_type(lu | hu, jnp.float32)
    for q in range(4):
        o_ref[:, WPR * q:WPR * (q + 1)] = wv[QB * q:QB * (q + 1), :]


def _tc_pack(ct, grid):
    return pl.pallas_call(
        _tc_pack_body,
        grid=(grid,),
        in_specs=[pl.BlockSpec((DIM, VB), lambda i: (0, i))],
        out_specs=pl.BlockSpec((QB, 4 * WPR), lambda i: (i, 0)),
        out_shape=jax.ShapeDtypeStruct((grid * QB, 4 * WPR), jnp.float32),
        compiler_params=pltpu.CompilerParams(
            dimension_semantics=("arbitrary",)),
    )(ct)


def _make_sc_body(nchunk):
    def body(tgt_i, ctx_i, tgt_tab, ctx_tab, out,
             tgt_idx_v, ctx_idx_v,
             tgt_a, tgt_b, ctx_a, ctx_b, out_v, sem_a, sem_b):
        w = lax.axis_index("s") * NC + lax.axis_index("c")
        lanes = lax.iota(jnp.int32, 16)
        pltpu.sync_copy(tgt_i.at[w], tgt_idx_v)
        pltpu.sync_copy(ctx_i.at[w], ctx_idx_v)

        tgt_bufs = (tgt_a, tgt_b)
        ctx_bufs = (ctx_a, ctx_b)
        sems = (sem_a, sem_b)

        def fire(k):
            par = k % 2
            waits = [pltpu.async_copy(
                tgt_tab.at[tgt_idx_v.at[k]], tgt_bufs[par], sems[par])]
            for c in range(NUM_CTX):
                waits.append(pltpu.async_copy(
                    ctx_tab.at[ctx_idx_v.at[k, c]],
                    ctx_bufs[par].at[pl.ds(c * CB, CB)], sems[par]))
            return waits

        def unpack2(row_ref, r):
            a0, b0 = plsc.unpack(plsc.bitcast(row_ref[r, pl.ds(0, 16)],
                                              jnp.bfloat16),
                                 format=plsc.PackFormat.INTERLEAVED)
            a1, b1 = plsc.unpack(plsc.bitcast(row_ref[r, pl.ds(16, 16)],
                                              jnp.bfloat16),
                                 format=plsc.PackFormat.INTERLEAVED)
            return (a0, a1, b0, b1)

        pending = fire(0)
        for k in range(nchunk):
            for h in pending:
                h.wait()
            if k + 1 < nchunk:
                pending = fire(k + 1)
            tgt_rows = tgt_bufs[k % 2]
            ctx_rows = ctx_bufs[k % 2]
            ks = jnp.full((16,), k, jnp.int32)

            def bstep(b, carry, k=k, tgt_rows=tgt_rows, ctx_rows=ctx_rows,
                      ks=ks):
                wv = [tgt_rows[b, pl.ds(16 * i, 16)] for i in range(4)]
                vec = jnp.zeros((16,), jnp.float32)
                for s in range(NUM_CTX):
                    p = b * NUM_CTX + s
                    xv = unpack2(ctx_rows, p)
                    acc = wv[0] * xv[0]
                    for i in range(1, 4):
                        acc = acc + wv[i] * xv[i]
                    vec = jnp.where(lanes == s, jnp.sum(acc), vec)
                plsc.store_scatter(out_v, [ks, b * NUM_CTX + lanes], vec,
                                   mask=lanes < NUM_CTX)
                return carry

            lax.fori_loop(0, CB, bstep, 0)

        pltpu.sync_copy(out_v, out.at[w])

    return body


def kernel(target, context, target_table, context_table):
    batch, num_ctx = context.shape
    vocab = target_table.shape[0]
    assert num_ctx == NUM_CTX and batch % (NW * CB) == 0
    nchunk = batch // (NW * CB)
    ppc = CB * NUM_CTX
    grid = (vocab + VB - 1) // VB

    cpk = _tc_pack(context_table.T, grid).reshape(4 * grid * QB, WPR)

    sh_vb = VB.bit_length() - 1
    sh_qb = QB.bit_length() - 1

    def addr(v):
        v = v.astype(jnp.int32)
        return (v >> sh_vb) * VB + (v & (QB - 1)) * 4 + ((v >> sh_qb) & 3)

    tgt_i = target.astype(jnp.int32).reshape(NW, nchunk, CB)
    ctx_i = addr(context).reshape(NW, nchunk, NUM_CTX, CB)

    mesh = plsc.VectorSubcoreMesh(core_axis_name="c", subcore_axis_name="s")
    grid_kernel = pl.kernel(
        _make_sc_body(nchunk),
        out_type=jax.ShapeDtypeStruct((NW, nchunk, ppc), jnp.float32),
        mesh=mesh,
        scratch_types=[
            pltpu.VMEM((nchunk, CB), jnp.int32),
            pltpu.VMEM((nchunk, NUM_CTX, CB), jnp.int32),
            pltpu.VMEM((CB, DIM), jnp.float32),
            pltpu.VMEM((CB, DIM), jnp.float32),
            pltpu.VMEM((NUM_CTX * CB, WPR), jnp.float32),
            pltpu.VMEM((NUM_CTX * CB, WPR), jnp.float32),
            pltpu.VMEM((nchunk, ppc), jnp.float32),
            pltpu.SemaphoreType.DMA,
            pltpu.SemaphoreType.DMA,
        ],
        compiler_params=pltpu.CompilerParams(
            needs_layout_passes=False, use_tc_tiling_on_sc=False),
    )
    out = grid_kernel(tgt_i, ctx_i, target_table, cpk)
    return out.reshape(batch, NUM_CTX)
